# baseline (speedup 1.0000x reference)
import functools

import jax
import jax.numpy as jnp
from jax import lax
from jax.experimental import pallas as pl
from jax.experimental.pallas import tpu as pltpu
from jax.experimental.pallas import tpu_sc as plsc

EMBED_DIM = 64
VB = 16384
CHUNK = 320
NBUF = 4


@functools.lru_cache(maxsize=None)
def _build_repack_tc(V: int):
    grid = (V + VB - 1) // VB

    def body(x_ref, o_ref):
        x3 = x_ref[...].T.reshape(VB // 2, 2, EMBED_DIM)
        o_ref[:, 0:EMBED_DIM] = x3[:, 0, :]
        o_ref[:, EMBED_DIM:2 * EMBED_DIM] = x3[:, 1, :]

    return pl.pallas_call(
        body,
        grid=(grid,),
        in_specs=[pl.BlockSpec((EMBED_DIM, VB), lambda i: (0, i))],
        out_specs=pl.BlockSpec((VB // 2, 2 * EMBED_DIM), lambda i: (i, 0)),
        out_shape=jax.ShapeDtypeStruct((V // 2, 2 * EMBED_DIM), jnp.float32),
    )


@functools.lru_cache(maxsize=None)
def _build_gather(B: int, V: int):
    info = plsc.get_sparse_core_info()
    NC, NS = info.num_cores, info.num_subcores
    NW = NC * NS
    b_per_w = B // NW
    nsteps = b_per_w // CHUNK
    assert B % NW == 0 and b_per_w % CHUNK == 0 and nsteps % NBUF == 0
    mesh = plsc.VectorSubcoreMesh(core_axis_name="c", subcore_axis_name="s")

    scratch = [pltpu.VMEM((b_per_w,), jnp.int32)]
    scratch += [pltpu.VMEM((CHUNK, EMBED_DIM), jnp.float32)
                for _ in range(NBUF)]
    scratch += [pltpu.SemaphoreType.DMA for _ in range(2 * NBUF)]

    @functools.partial(
        pl.kernel,
        mesh=mesh,
        out_type=jax.ShapeDtypeStruct((B, 2 * EMBED_DIM), jnp.float32),
        scratch_types=scratch,
        compiler_params=pltpu.CompilerParams(use_tc_tiling_on_sc=False),
    )
    def gather_kernel(idx_hbm, table_hbm, out_hbm, idx_v, *rest):
        rows = rest[:NBUF]
        sem_g = rest[NBUF:2 * NBUF]
        sem_w = rest[2 * NBUF:]
        wid = lax.axis_index("s") * NC + lax.axis_index("c")
        bbase = wid * b_per_w

        pltpu.sync_copy(idx_hbm.at[pl.ds(bbase, b_per_w)], idx_v)

        def start_g(s, b):
            pltpu.async_copy(
                table_hbm.at[idx_v.at[pl.ds(s * CHUNK, CHUNK)]], rows[b],
                sem_g[b])

        def wait_g(b):
            pltpu.make_async_copy(
                table_hbm.at[idx_v.at[pl.ds(0, CHUNK)]], rows[b],
                sem_g[b]).wait()

        def start_w(s, b):
            pltpu.async_copy(
                rows[b],
                out_hbm.at[pl.ds(bbase + s * CHUNK, CHUNK),
                           pl.ds(0, EMBED_DIM)], sem_w[b])

        def wait_w(b):
            pltpu.make_async_copy(
                rows[b],
                out_hbm.at[pl.ds(bbase, CHUNK), pl.ds(0, EMBED_DIM)],
                sem_w[b]).wait()

        start_g(0, 0)
        start_g(1, 1)

        def outer(t, carry):
            for b in range(NBUF):
                s = t * NBUF + b
                wait_g(b)
                start_w(s, b)
                b2 = (b + 2) % NBUF

                @pl.when(s + 2 < nsteps)
                def _issue():
                    @pl.when(s >= 2)
                    def _drain():
                        wait_w(b2)
                    start_g(s + 2, b2)
            return carry

        lax.fori_loop(0, nsteps // NBUF, outer, 0)
        for b in range(NBUF):
            wait_w(b)

    return gather_kernel


def kernel(input, hidden, table):
    BATCH, HIST = input.shape
    V, E = table.shape
    B = BATCH * HIST
    packed = _build_repack_tc(V)(table.T)
    t_rm = packed.reshape(V, E)
    idx = input.T.reshape(B).astype(jnp.int32)
    out = _build_gather(B, V)(idx, t_rm)
    return out.reshape(HIST, BATCH, 2 * E)[:, :, :E].transpose(1, 0, 2)
